# baseline (speedup 1.0000x reference)
import functools

import jax
import jax.numpy as jnp
from jax import lax
from jax.experimental import pallas as pl
from jax.experimental.pallas import tpu as pltpu
from jax.experimental.pallas import tpu_sc as plsc

N = 10000
E = 320000
D = 128
H = 64
G = 64

NC, NS = 2, 16
NW = NC * NS
EPT = E // NW
CH = 125
NCH = EPT // CH
NPAD = 10240
RPT = NPAD // NS

_mesh = plsc.VectorSubcoreMesh(core_axis_name="c", subcore_axis_name="s")
_f32 = jnp.float32
_sc_params = pltpu.CompilerParams(use_tc_tiling_on_sc=False)


def _ids():
    cid = lax.axis_index("c")
    sid = lax.axis_index("s")
    return cid * NS + sid, sid, cid


@functools.partial(
    pl.kernel,
    out_type=jax.ShapeDtypeStruct((NC, NPAD), _f32),
    mesh=_mesh,
    compiler_params=_sc_params,
    scratch_types=[
        pltpu.VMEM((NCH, CH), jnp.int32),
        pltpu.VMEM((CH,), _f32),
        pltpu.VMEM_SHARED((NPAD,), _f32),
        pltpu.SemaphoreType.DMA,
    ],
)
def _deg_kernel(edges4d_hbm, zn_hbm, out_hbm, idxd_v, ones_v, deg_sh, dsem):
    wid, sid, cid = _ids()
    pltpu.sync_copy(zn_hbm, deg_sh.at[pl.ds(sid * RPT, RPT)])
    pltpu.sync_copy(edges4d_hbm.at[1, wid], idxd_v)
    for k in range(0, CH - 15, 16):
        ones_v[pl.ds(k, 16)] = jnp.ones((16,), _f32)
    ones_v[pl.ds(CH - 16, 16)] = jnp.ones((16,), _f32)
    plsc.subcore_barrier()

    def body(j, carry):
        pltpu.async_copy(ones_v, deg_sh.at[idxd_v.at[j]], dsem, add=True)
        return carry

    lax.fori_loop(0, NCH, body, 0)

    def drain(j, carry):
        pltpu.make_async_copy(ones_v, deg_sh.at[idxd_v.at[j]], dsem).wait()
        return carry

    lax.fori_loop(0, NCH, drain, 0)
    plsc.subcore_barrier()
    pltpu.sync_copy(deg_sh.at[pl.ds(sid * RPT, RPT)],
                    out_hbm.at[cid, pl.ds(sid * RPT, RPT)])


@functools.partial(
    pl.kernel,
    out_type=jax.ShapeDtypeStruct((NC, NPAD, H), _f32),
    mesh=_mesh,
    compiler_params=_sc_params,
    scratch_types=[
        pltpu.VMEM((NCH, CH), jnp.int32),
        pltpu.VMEM((NCH, CH), jnp.int32),
        [pltpu.VMEM((CH, H), _f32)] * 4,
        pltpu.VMEM_SHARED((NPAD, H), _f32),
        [pltpu.SemaphoreType.DMA] * 4,
        [pltpu.SemaphoreType.DMA] * 4,
    ],
)
def _agg_kernel(hs_hbm, edges4d_hbm, zrows_hbm, out_hbm,
                idxs_v, idxd_v, bufs, acc_sh, gsems, ssems):
    wid, sid, cid = _ids()
    pltpu.sync_copy(zrows_hbm, acc_sh.at[pl.ds(sid * RPT, RPT)])
    pltpu.sync_copy(edges4d_hbm.at[0, wid], idxs_v)
    pltpu.sync_copy(edges4d_hbm.at[1, wid], idxd_v)
    plsc.subcore_barrier()

    def wait_gather(j, b):
        pltpu.make_async_copy(hs_hbm.at[idxs_v.at[j]], bufs[b],
                              gsems[b]).wait()

    def wait_scatter(j, b):
        pltpu.make_async_copy(bufs[b], acc_sh.at[idxd_v.at[j]],
                              ssems[b]).wait()

    pltpu.async_copy(hs_hbm.at[idxs_v.at[0]], bufs[0], gsems[0])
    pltpu.async_copy(hs_hbm.at[idxs_v.at[1]], bufs[1], gsems[1])

    def outer(g, carry):
        for b in range(4):
            j = g * 4 + b
            b2 = (b + 2) % 4
            wait_gather(j, b)
            pltpu.async_copy(bufs[b], acc_sh.at[idxd_v.at[j]], ssems[b],
                             add=True)

            @pl.when(j >= 2)
            def _ws():
                wait_scatter(j - 2, b2)

            @pl.when(j + 2 < NCH)
            def _issue():
                pltpu.async_copy(hs_hbm.at[idxs_v.at[j + 2]], bufs[b2],
                                 gsems[b2])
        return carry

    lax.fori_loop(0, NCH // 4, outer, 0)
    wait_scatter(NCH - 2, (NCH - 2) % 4)
    wait_scatter(NCH - 1, (NCH - 1) % 4)
    plsc.subcore_barrier()
    pltpu.sync_copy(acc_sh.at[pl.ds(sid * RPT, RPT)],
                    out_hbm.at[cid, pl.ds(sid * RPT, RPT)])


def _mm1_body(x_ref, w1_ref, h1raw_ref):
    h1raw_ref[...] = jnp.dot(x_ref[...], w1_ref[...],
                             preferred_element_type=_f32)


def _scale_body(degp_ref, h1raw_ref, hs1_ref, dinv_ref):
    deg = degp_ref[0, pl.ds(0, N)] + degp_ref[1, pl.ds(0, N)] + 1.0
    dinv = lax.rsqrt(deg)
    dinv_ref[...] = dinv
    hs1_ref[...] = h1raw_ref[...] * dinv[:, None]


def _mm2_body(accp_ref, hs1_ref, dinv_ref, w2_ref, b1_ref, hs2_ref):
    dinv = dinv_ref[...]
    acc = accp_ref[0, pl.ds(0, N), :] + accp_ref[1, pl.ds(0, N), :]
    tot = (acc + hs1_ref[...]) * dinv[:, None]
    h1 = jnp.maximum(tot + b1_ref[...][None, :], 0.0)
    hs2_ref[...] = jnp.dot(h1, w2_ref[...], preferred_element_type=_f32) * dinv[:, None]


def _fin_body(accp_ref, hs2_ref, dinv_ref, b2_ref, batch_ref, wf_ref, bf_ref,
              out_ref):
    dinv = dinv_ref[...]
    acc = accp_ref[0, pl.ds(0, N), :] + accp_ref[1, pl.ds(0, N), :]
    tot = (acc + hs2_ref[...]) * dinv[:, None]
    h2 = jnp.maximum(tot + b2_ref[...][None, :], 0.0)
    b = batch_ref[...]
    onehot_t = (b[None, :] == lax.broadcasted_iota(jnp.int32, (G, N), 0))
    onehot_t = onehot_t.astype(_f32)
    sums = jnp.dot(onehot_t, h2, preferred_element_type=_f32)
    counts = jnp.sum(onehot_t, axis=1)
    pooled = sums / jnp.maximum(counts, 1.0)[:, None]
    z = jnp.dot(pooled, wf_ref[...], preferred_element_type=_f32) + bf_ref[...][None, :]
    out_ref[...] = 1.0 / (1.0 + jnp.exp(-z))


def kernel(x, edge_index, batch, W1, b1, W2, b2, Wf, bf):
    edges4d = edge_index.reshape(2, NW, NCH, CH)
    zn = jnp.zeros((RPT,), _f32)
    zrows = jnp.zeros((RPT, H), _f32)

    degp = _deg_kernel(edges4d, zn)
    h1raw = pl.pallas_call(
        _mm1_body,
        out_shape=jax.ShapeDtypeStruct((N, H), _f32),
    )(x, W1)
    hs1, dinv = pl.pallas_call(
        _scale_body,
        out_shape=[jax.ShapeDtypeStruct((N, H), _f32),
                   jax.ShapeDtypeStruct((N,), _f32)],
    )(degp, h1raw)
    acc1 = _agg_kernel(hs1, edges4d, zrows)
    hs2 = pl.pallas_call(
        _mm2_body,
        out_shape=jax.ShapeDtypeStruct((N, H), _f32),
    )(acc1, hs1, dinv, W2, b1)
    acc2 = _agg_kernel(hs2, edges4d, zrows)
    out = pl.pallas_call(
        _fin_body,
        out_shape=jax.ShapeDtypeStruct((G, 1), _f32),
    )(acc2, hs2, dinv, b2, batch, Wf, bf)
    return out


# speedup vs baseline: 1.0596x; 1.0596x over previous
import functools

import jax
import jax.numpy as jnp
from jax import lax
from jax.experimental import pallas as pl
from jax.experimental.pallas import tpu as pltpu
from jax.experimental.pallas import tpu_sc as plsc

N = 10000
E = 320000
D = 128
H = 64
G = 64

NC, NS = 2, 16
NW = NC * NS
EPT = E // NW
CH = 125
NCH = EPT // CH
NPAD = 10240
RPT = NPAD // NS

_mesh = plsc.VectorSubcoreMesh(core_axis_name="c", subcore_axis_name="s")
_f32 = jnp.float32
_sc_params = pltpu.CompilerParams(use_tc_tiling_on_sc=False)


def _ids():
    cid = lax.axis_index("c")
    sid = lax.axis_index("s")
    return cid * NS + sid, sid, cid


@functools.partial(
    pl.kernel,
    out_type=jax.ShapeDtypeStruct((NC, NPAD), _f32),
    mesh=_mesh,
    compiler_params=_sc_params,
    scratch_types=[
        pltpu.VMEM((NCH, CH), jnp.int32),
        pltpu.VMEM((CH,), _f32),
        pltpu.VMEM_SHARED((NPAD,), _f32),
        pltpu.SemaphoreType.DMA,
    ],
)
def _deg_kernel(edges4d_hbm, zn_hbm, out_hbm, idxd_v, ones_v, deg_sh, dsem):
    wid, sid, cid = _ids()
    pltpu.sync_copy(zn_hbm, deg_sh.at[pl.ds(sid * RPT, RPT)])
    pltpu.sync_copy(edges4d_hbm.at[1, wid], idxd_v)
    for k in range(0, CH - 15, 16):
        ones_v[pl.ds(k, 16)] = jnp.ones((16,), _f32)
    ones_v[pl.ds(CH - 16, 16)] = jnp.ones((16,), _f32)
    plsc.subcore_barrier()

    def body(j, carry):
        pltpu.async_copy(ones_v, deg_sh.at[idxd_v.at[j]], dsem, add=True)
        return carry

    lax.fori_loop(0, NCH, body, 0)

    def drain(j, carry):
        pltpu.make_async_copy(ones_v, deg_sh.at[idxd_v.at[j]], dsem).wait()
        return carry

    lax.fori_loop(0, NCH, drain, 0)
    plsc.subcore_barrier()
    pltpu.sync_copy(deg_sh.at[pl.ds(sid * RPT, RPT)],
                    out_hbm.at[cid, pl.ds(sid * RPT, RPT)])


@functools.partial(
    pl.kernel,
    out_type=jax.ShapeDtypeStruct((NC, NPAD, H), _f32),
    mesh=_mesh,
    compiler_params=_sc_params,
    scratch_types=[
        pltpu.VMEM((NCH, CH), jnp.int32),
        pltpu.VMEM((NCH, CH), jnp.int32),
        [pltpu.VMEM((CH, H), _f32)] * 6,
        pltpu.VMEM_SHARED((NPAD, H), _f32),
        [pltpu.SemaphoreType.DMA] * 6,
        [pltpu.SemaphoreType.DMA] * 6,
    ],
)
def _agg_kernel(hs_hbm, edges4d_hbm, zrows_hbm, out_hbm,
                idxs_v, idxd_v, bufs, acc_sh, gsems, ssems):
    wid, sid, cid = _ids()
    pltpu.sync_copy(zrows_hbm, acc_sh.at[pl.ds(sid * RPT, RPT)])
    pltpu.sync_copy(edges4d_hbm.at[0, wid], idxs_v)
    pltpu.sync_copy(edges4d_hbm.at[1, wid], idxd_v)
    plsc.subcore_barrier()

    def wait_gather(j, b):
        pltpu.make_async_copy(hs_hbm.at[idxs_v.at[j]], bufs[b],
                              gsems[b]).wait()

    def wait_scatter(j, b):
        pltpu.make_async_copy(bufs[b], acc_sh.at[idxd_v.at[j]],
                              ssems[b]).wait()

    for p in range(3):
        pltpu.async_copy(hs_hbm.at[idxs_v.at[p]], bufs[p], gsems[p])

    def outer(g, carry):
        for b in range(6):
            j = g * 6 + b
            b2 = (b + 3) % 6
            wait_gather(j, b)
            pltpu.async_copy(bufs[b], acc_sh.at[idxd_v.at[j]], ssems[b],
                             add=True)

            @pl.when(j >= 3)
            def _ws():
                wait_scatter(j - 3, b2)

            @pl.when(j + 3 < NCH)
            def _issue():
                pltpu.async_copy(hs_hbm.at[idxs_v.at[j + 3]], bufs[b2],
                                 gsems[b2])
        return carry

    lax.fori_loop(0, NCH // 6, outer, 0)
    for j in range(78, NCH):
        b = j % 6
        wait_gather(j, b)
        pltpu.async_copy(bufs[b], acc_sh.at[idxd_v.at[j]], ssems[b], add=True)
        wait_scatter(j - 3, (j + 3) % 6)
    for j in range(NCH - 3, NCH):
        wait_scatter(j, j % 6)
    plsc.subcore_barrier()
    pltpu.sync_copy(acc_sh.at[pl.ds(sid * RPT, RPT)],
                    out_hbm.at[cid, pl.ds(sid * RPT, RPT)])


def _mm1_body(degp_ref, x_ref, w1_ref, hs1_ref, dinv_ref):
    deg = degp_ref[0, pl.ds(0, N)] + degp_ref[1, pl.ds(0, N)] + 1.0
    dinv = lax.rsqrt(deg)
    dinv_ref[...] = dinv
    h = jnp.dot(x_ref[...], w1_ref[...], preferred_element_type=_f32)
    hs1_ref[...] = h * dinv[:, None]


def _mm2_body(accp_ref, hs1_ref, dinv_ref, w2_ref, b1_ref, hs2_ref):
    dinv = dinv_ref[...]
    acc = accp_ref[0, pl.ds(0, N), :] + accp_ref[1, pl.ds(0, N), :]
    tot = (acc + hs1_ref[...]) * dinv[:, None]
    h1 = jnp.maximum(tot + b1_ref[...][None, :], 0.0)
    hs2_ref[...] = jnp.dot(h1, w2_ref[...], preferred_element_type=_f32) * dinv[:, None]


def _fin_body(accp_ref, hs2_ref, dinv_ref, b2_ref, batch_ref, wf_ref, bf_ref,
              out_ref):
    dinv = dinv_ref[...]
    acc = accp_ref[0, pl.ds(0, N), :] + accp_ref[1, pl.ds(0, N), :]
    tot = (acc + hs2_ref[...]) * dinv[:, None]
    h2 = jnp.maximum(tot + b2_ref[...][None, :], 0.0)
    b = batch_ref[...]
    onehot_t = (b[None, :] == lax.broadcasted_iota(jnp.int32, (G, N), 0))
    onehot_t = onehot_t.astype(_f32)
    sums = jnp.dot(onehot_t, h2, preferred_element_type=_f32)
    counts = jnp.sum(onehot_t, axis=1)
    pooled = sums / jnp.maximum(counts, 1.0)[:, None]
    z = jnp.dot(pooled, wf_ref[...], preferred_element_type=_f32) + bf_ref[...][None, :]
    out_ref[...] = 1.0 / (1.0 + jnp.exp(-z))


def kernel(x, edge_index, batch, W1, b1, W2, b2, Wf, bf):
    edges4d = edge_index.reshape(2, NW, NCH, CH)
    zn = jnp.zeros((RPT,), _f32)
    zrows = jnp.zeros((RPT, H), _f32)

    degp = _deg_kernel(edges4d, zn)
    hs1, dinv = pl.pallas_call(
        _mm1_body,
        out_shape=[jax.ShapeDtypeStruct((N, H), _f32),
                   jax.ShapeDtypeStruct((N,), _f32)],
    )(degp, x, W1)
    acc1 = _agg_kernel(hs1, edges4d, zrows)
    hs2 = pl.pallas_call(
        _mm2_body,
        out_shape=jax.ShapeDtypeStruct((N, H), _f32),
    )(acc1, hs1, dinv, W2, b1)
    acc2 = _agg_kernel(hs2, edges4d, zrows)
    out = pl.pallas_call(
        _fin_body,
        out_shape=jax.ShapeDtypeStruct((G, 1), _f32),
    )(acc2, hs2, dinv, b2, batch, Wf, bf)
    return out


# speedup vs baseline: 1.0785x; 1.0179x over previous
import functools

import jax
import jax.numpy as jnp
from jax import lax
from jax.experimental import pallas as pl
from jax.experimental.pallas import tpu as pltpu
from jax.experimental.pallas import tpu_sc as plsc

N = 10000
E = 320000
D = 128
H = 64
G = 64

NC, NS = 2, 16
NW = NC * NS
EPT = E // NW
CH = 125
NCH = EPT // CH
NPAD = 10240
RPT = NPAD // NS

_mesh = plsc.VectorSubcoreMesh(core_axis_name="c", subcore_axis_name="s")
_f32 = jnp.float32
_sc_params = pltpu.CompilerParams(use_tc_tiling_on_sc=False)


def _ids():
    cid = lax.axis_index("c")
    sid = lax.axis_index("s")
    return cid * NS + sid, sid, cid


@functools.partial(
    pl.kernel,
    out_type=jax.ShapeDtypeStruct((NC, NPAD), _f32),
    mesh=_mesh,
    compiler_params=_sc_params,
    scratch_types=[
        pltpu.VMEM((NCH, CH), jnp.int32),
        pltpu.VMEM((CH,), _f32),
        pltpu.VMEM_SHARED((NPAD,), _f32),
        pltpu.SemaphoreType.DMA,
    ],
)
def _deg_kernel(edges4d_hbm, zn_hbm, out_hbm, idxd_v, ones_v, deg_sh, dsem):
    wid, sid, cid = _ids()
    pltpu.sync_copy(zn_hbm, deg_sh.at[pl.ds(sid * RPT, RPT)])
    pltpu.sync_copy(edges4d_hbm.at[1, wid], idxd_v)
    for k in range(0, CH - 15, 16):
        ones_v[pl.ds(k, 16)] = jnp.ones((16,), _f32)
    ones_v[pl.ds(CH - 16, 16)] = jnp.ones((16,), _f32)
    plsc.subcore_barrier()

    def body(j, carry):
        pltpu.async_copy(ones_v, deg_sh.at[idxd_v.at[j]], dsem, add=True)
        return carry

    lax.fori_loop(0, NCH, body, 0)

    def drain(j, carry):
        pltpu.make_async_copy(ones_v, deg_sh.at[idxd_v.at[j]], dsem).wait()
        return carry

    lax.fori_loop(0, NCH, drain, 0)
    plsc.subcore_barrier()
    pltpu.sync_copy(deg_sh.at[pl.ds(sid * RPT, RPT)],
                    out_hbm.at[cid, pl.ds(sid * RPT, RPT)])


@functools.partial(
    pl.kernel,
    out_type=jax.ShapeDtypeStruct((NC, NPAD, H), _f32),
    mesh=_mesh,
    compiler_params=_sc_params,
    scratch_types=[
        pltpu.VMEM((NCH, CH), jnp.int32),
        pltpu.VMEM((NCH, CH), jnp.int32),
        [pltpu.VMEM((CH, H), _f32)] * 8,
        pltpu.VMEM_SHARED((NPAD, H), _f32),
        [pltpu.SemaphoreType.DMA] * 8,
        [pltpu.SemaphoreType.DMA] * 8,
    ],
)
def _agg_kernel(hs_hbm, edges4d_hbm, zrows_hbm, out_hbm,
                idxs_v, idxd_v, bufs, acc_sh, gsems, ssems):
    wid, sid, cid = _ids()
    pltpu.sync_copy(zrows_hbm, acc_sh.at[pl.ds(sid * RPT, RPT)])
    pltpu.sync_copy(edges4d_hbm.at[0, wid], idxs_v)
    pltpu.sync_copy(edges4d_hbm.at[1, wid], idxd_v)
    plsc.subcore_barrier()

    def wait_gather(j, b):
        pltpu.make_async_copy(hs_hbm.at[idxs_v.at[j]], bufs[b],
                              gsems[b]).wait()

    def wait_scatter(j, b):
        pltpu.make_async_copy(bufs[b], acc_sh.at[idxd_v.at[j]],
                              ssems[b]).wait()

    for p in range(4):
        pltpu.async_copy(hs_hbm.at[idxs_v.at[p]], bufs[p], gsems[p])

    def outer(g, carry):
        for b in range(8):
            j = g * 8 + b
            b2 = (b + 4) % 8
            wait_gather(j, b)
            pltpu.async_copy(bufs[b], acc_sh.at[idxd_v.at[j]], ssems[b],
                             add=True)

            @pl.when(j >= 4)
            def _ws():
                wait_scatter(j - 4, b2)

            @pl.when(j + 4 < NCH)
            def _issue():
                pltpu.async_copy(hs_hbm.at[idxs_v.at[j + 4]], bufs[b2],
                                 gsems[b2])
        return carry

    lax.fori_loop(0, NCH // 8, outer, 0)
    for j in range(NCH - 4, NCH):
        wait_scatter(j, j % 8)
    plsc.subcore_barrier()
    pltpu.sync_copy(acc_sh.at[pl.ds(sid * RPT, RPT)],
                    out_hbm.at[cid, pl.ds(sid * RPT, RPT)])


def _mm1_body(degp_ref, x_ref, w1_ref, hs1_ref, dinv_ref):
    deg = degp_ref[0, pl.ds(0, N)] + degp_ref[1, pl.ds(0, N)] + 1.0
    dinv = lax.rsqrt(deg)
    dinv_ref[...] = dinv
    h = jnp.dot(x_ref[...], w1_ref[...], preferred_element_type=_f32)
    hs1_ref[...] = h * dinv[:, None]


def _mm2_body(accp_ref, hs1_ref, dinv_ref, w2_ref, b1_ref, hs2_ref):
    dinv = dinv_ref[...]
    acc = accp_ref[0, pl.ds(0, N), :] + accp_ref[1, pl.ds(0, N), :]
    tot = (acc + hs1_ref[...]) * dinv[:, None]
    h1 = jnp.maximum(tot + b1_ref[...][None, :], 0.0)
    hs2_ref[...] = jnp.dot(h1, w2_ref[...], preferred_element_type=_f32) * dinv[:, None]


def _fin_body(accp_ref, hs2_ref, dinv_ref, b2_ref, batch_ref, wf_ref, bf_ref,
              out_ref):
    dinv = dinv_ref[...]
    acc = accp_ref[0, pl.ds(0, N), :] + accp_ref[1, pl.ds(0, N), :]
    tot = (acc + hs2_ref[...]) * dinv[:, None]
    h2 = jnp.maximum(tot + b2_ref[...][None, :], 0.0)
    b = batch_ref[...]
    onehot_t = (b[None, :] == lax.broadcasted_iota(jnp.int32, (G, N), 0))
    onehot_t = onehot_t.astype(_f32)
    sums = jnp.dot(onehot_t, h2, preferred_element_type=_f32)
    counts = jnp.sum(onehot_t, axis=1)
    pooled = sums / jnp.maximum(counts, 1.0)[:, None]
    z = jnp.dot(pooled, wf_ref[...], preferred_element_type=_f32) + bf_ref[...][None, :]
    out_ref[...] = 1.0 / (1.0 + jnp.exp(-z))


def kernel(x, edge_index, batch, W1, b1, W2, b2, Wf, bf):
    edges4d = edge_index.reshape(2, NW, NCH, CH)
    zn = jnp.zeros((RPT,), _f32)
    zrows = jnp.zeros((RPT, H), _f32)

    degp = _deg_kernel(edges4d, zn)
    hs1, dinv = pl.pallas_call(
        _mm1_body,
        out_shape=[jax.ShapeDtypeStruct((N, H), _f32),
                   jax.ShapeDtypeStruct((N,), _f32)],
    )(degp, x, W1)
    acc1 = _agg_kernel(hs1, edges4d, zrows)
    hs2 = pl.pallas_call(
        _mm2_body,
        out_shape=jax.ShapeDtypeStruct((N, H), _f32),
    )(acc1, hs1, dinv, W2, b1)
    acc2 = _agg_kernel(hs2, edges4d, zrows)
    out = pl.pallas_call(
        _fin_body,
        out_shape=jax.ShapeDtypeStruct((G, 1), _f32),
    )(acc2, hs2, dinv, b2, batch, Wf, bf)
    return out
